# Initial kernel scaffold; baseline (speedup 1.0000x reference)
import jax
import jax.numpy as jnp
from jax.experimental import pallas as pl


def kernel(V, E_n, R_r, W1, b1, W2, b2, W3, b3):
    raise NotImplementedError("write your pallas kernel here")


# speedup vs baseline: 4.7137x; 4.7137x over previous
import functools

import jax
import jax.numpy as jnp
from jax import lax
from jax.experimental import pallas as pl
from jax.experimental.pallas import tpu as pltpu
from jax.experimental.pallas import tpu_sc as plsc

_NUM_TILES = 16
_CHUNK = 80


def _scatter_add_sc(E_n, idx4, n_nodes):
    b_dim, _, chunks_per_tile, chunk = idx4.shape
    d = E_n.shape[-1]
    node_stride = (n_nodes // _NUM_TILES) // 8 * 8
    node_size = n_nodes - (_NUM_TILES - 1) * node_stride

    mesh = plsc.VectorSubcoreMesh(core_axis_name="c", subcore_axis_name="s")

    @functools.partial(
        pl.kernel,
        mesh=mesh,
        out_type=jax.ShapeDtypeStruct((b_dim, n_nodes, d), jnp.float32),
        scratch_types=[
            pltpu.VMEM((chunks_per_tile, chunk), jnp.int32),
            pltpu.VMEM((chunk, d), jnp.float32),
            pltpu.VMEM_SHARED((n_nodes, d), jnp.float32),
        ],
    )
    def scat(en_hbm, idx_hbm, out_hbm, idx_v, eb, agg_sh):
        b = lax.axis_index("c")
        s = lax.axis_index("s")

        zeros16 = jnp.zeros((16,), jnp.float32)

        def zrow(r, carry):
            for k in range(d // 16):
                eb[r, pl.ds(k * 16, 16)] = zeros16
            return carry

        lax.fori_loop(0, chunk, zrow, 0)

        base_node = s * node_stride
        full = node_size // chunk
        rem = node_size - full * chunk
        for i in range(full):
            pltpu.sync_copy(eb, agg_sh.at[pl.ds(base_node + i * chunk, chunk)])
        if rem:
            pltpu.sync_copy(eb.at[pl.ds(0, rem)],
                            agg_sh.at[pl.ds(base_node + full * chunk, rem)])

        pltpu.sync_copy(idx_hbm.at[b, s], idx_v)

        plsc.subcore_barrier()

        def body(j, carry):
            e0 = (s * chunks_per_tile + j) * chunk
            pltpu.sync_copy(en_hbm.at[b, pl.ds(e0, chunk)], eb)
            pltpu.sync_copy(eb, agg_sh.at[idx_v.at[j]], add=True)
            return carry

        lax.fori_loop(0, chunks_per_tile, body, 0)

        plsc.subcore_barrier()

        pltpu.sync_copy(agg_sh.at[pl.ds(base_node, node_size)],
                        out_hbm.at[b, pl.ds(base_node, node_size)])

    return scat


def _mlp_body(v_ref, a_ref, w1v_ref, w1a_ref, b1_ref, w2_ref, b2_ref,
              w3_ref, b3_ref, o_ref):
    h = jnp.dot(v_ref[...], w1v_ref[...], preferred_element_type=jnp.float32)
    h = h + jnp.dot(a_ref[...], w1a_ref[...], preferred_element_type=jnp.float32)
    h = jnp.maximum(h + b1_ref[...], 0.0)
    h = jnp.maximum(
        jnp.dot(h, w2_ref[...], preferred_element_type=jnp.float32) + b2_ref[...], 0.0)
    o_ref[...] = jnp.maximum(
        jnp.dot(h, w3_ref[...], preferred_element_type=jnp.float32) + b3_ref[...], 0.0)


def kernel(V, E_n, R_r, W1, b1, W2, b2, W3, b3):
    b_dim, n_nodes, dv = V.shape
    e_edges, de = E_n.shape[1], E_n.shape[2]
    dout = W1.shape[1]

    idx = R_r[..., 0].astype(jnp.int32)
    idx4 = idx.reshape(b_dim, _NUM_TILES, e_edges // (_NUM_TILES * _CHUNK), _CHUNK)

    agg = _scatter_add_sc(E_n, idx4, n_nodes)(E_n, idx4)

    total = b_dim * n_nodes
    blk = 2000
    v2 = V.reshape(total, dv)
    a2 = agg.reshape(total, de)
    w1v, w1a = W1[:dv], W1[dv:]

    h = pl.pallas_call(
        _mlp_body,
        grid=(total // blk,),
        in_specs=[
            pl.BlockSpec((blk, dv), lambda i: (i, 0)),
            pl.BlockSpec((blk, de), lambda i: (i, 0)),
            pl.BlockSpec((dv, dout), lambda i: (0, 0)),
            pl.BlockSpec((de, dout), lambda i: (0, 0)),
            pl.BlockSpec((1, dout), lambda i: (0, 0)),
            pl.BlockSpec((dout, dout), lambda i: (0, 0)),
            pl.BlockSpec((1, dout), lambda i: (0, 0)),
            pl.BlockSpec((dout, dout), lambda i: (0, 0)),
            pl.BlockSpec((1, dout), lambda i: (0, 0)),
        ],
        out_specs=pl.BlockSpec((blk, dout), lambda i: (i, 0)),
        out_shape=jax.ShapeDtypeStruct((total, dout), jnp.float32),
    )(v2, a2, w1v, w1a, b1[None, :], W2, b2[None, :], W3, b3[None, :])

    return h.reshape(b_dim, n_nodes, dout)


# speedup vs baseline: 7.3905x; 1.5679x over previous
import functools

import jax
import jax.numpy as jnp
from jax import lax
from jax.experimental import pallas as pl
from jax.experimental.pallas import tpu as pltpu
from jax.experimental.pallas import tpu_sc as plsc

_NUM_TILES = 16
_CHUNK = 80


def _scatter_add_sc(E_n, idx4, n_nodes):
    b_dim, _, chunks_per_tile, chunk = idx4.shape
    d = E_n.shape[-1]
    node_stride = (n_nodes // _NUM_TILES) // 8 * 8
    node_size = n_nodes - (_NUM_TILES - 1) * node_stride

    mesh = plsc.VectorSubcoreMesh(core_axis_name="c", subcore_axis_name="s")

    edges_per_tile = chunks_per_tile * chunk

    @functools.partial(
        pl.kernel,
        mesh=mesh,
        out_type=jax.ShapeDtypeStruct((b_dim, n_nodes, d), jnp.float32),
        scratch_types=[
            pltpu.VMEM((chunks_per_tile, chunk), jnp.int32),
            pltpu.VMEM((chunk, d), jnp.float32),
            pltpu.VMEM((chunk, d), jnp.float32),
            pltpu.VMEM_SHARED((n_nodes, d), jnp.float32),
            pltpu.SemaphoreType.DMA,
            pltpu.SemaphoreType.DMA,
        ],
    )
    def scat(en_hbm, idx_hbm, out_hbm, idx_v, eb_a, eb_b, agg_sh,
             gsem_a, gsem_b):
        b = lax.axis_index("c")
        s = lax.axis_index("s")
        base_edge = s * edges_per_tile
        base_node = s * node_stride

        def gather(j, eb, sem):
            return pltpu.make_async_copy(
                en_hbm.at[b, pl.ds(base_edge + j * chunk, chunk)], eb, sem)

        def scatter(j, eb):
            pltpu.sync_copy(eb, agg_sh.at[idx_v.at[j]], add=True)

        zeros16 = jnp.zeros((16,), jnp.float32)

        def zrow(r, carry):
            for k in range(d // 16):
                eb_a[r, pl.ds(k * 16, 16)] = zeros16
            return carry

        lax.fori_loop(0, chunk, zrow, 0)

        full = node_size // chunk
        for i in range(full):
            pltpu.sync_copy(eb_a, agg_sh.at[pl.ds(base_node + i * chunk, chunk)])
        pltpu.sync_copy(idx_hbm.at[b, s], idx_v)

        plsc.subcore_barrier()

        gather(0, eb_a, gsem_a).start()

        def body(i, carry):
            g = 2 * i
            gather(g + 1, eb_b, gsem_b).start()
            gather(g, eb_a, gsem_a).wait()
            scatter(g, eb_a)
            gather(g + 2, eb_a, gsem_a).start()
            gather(g + 1, eb_b, gsem_b).wait()
            scatter(g + 1, eb_b)
            return carry

        lax.fori_loop(0, (chunks_per_tile - 1) // 2, body, 0)
        gather(chunks_per_tile - 1, eb_a, gsem_a).wait()
        scatter(chunks_per_tile - 1, eb_a)

        plsc.subcore_barrier()

        pltpu.sync_copy(agg_sh.at[pl.ds(base_node, node_size)],
                        out_hbm.at[b, pl.ds(base_node, node_size)])

    return scat


def _mlp_body(v_ref, a_ref, w1v_ref, w1a_ref, b1_ref, w2_ref, b2_ref,
              w3_ref, b3_ref, o_ref):
    h = jnp.dot(v_ref[...], w1v_ref[...], preferred_element_type=jnp.float32)
    h = h + jnp.dot(a_ref[...], w1a_ref[...], preferred_element_type=jnp.float32)
    h = jnp.maximum(h + b1_ref[...], 0.0)
    h = jnp.maximum(
        jnp.dot(h, w2_ref[...], preferred_element_type=jnp.float32) + b2_ref[...], 0.0)
    o_ref[...] = jnp.maximum(
        jnp.dot(h, w3_ref[...], preferred_element_type=jnp.float32) + b3_ref[...], 0.0)


def kernel(V, E_n, R_r, W1, b1, W2, b2, W3, b3):
    b_dim, n_nodes, dv = V.shape
    e_edges, de = E_n.shape[1], E_n.shape[2]
    dout = W1.shape[1]

    idx = R_r[..., 0].astype(jnp.int32)
    idx4 = idx.reshape(b_dim, _NUM_TILES, e_edges // (_NUM_TILES * _CHUNK), _CHUNK)

    agg = _scatter_add_sc(E_n, idx4, n_nodes)(E_n, idx4)

    total = b_dim * n_nodes
    blk = 2000
    v2 = V.reshape(total, dv)
    a2 = agg.reshape(total, de)
    w1v, w1a = W1[:dv], W1[dv:]

    h = pl.pallas_call(
        _mlp_body,
        grid=(total // blk,),
        in_specs=[
            pl.BlockSpec((blk, dv), lambda i: (i, 0)),
            pl.BlockSpec((blk, de), lambda i: (i, 0)),
            pl.BlockSpec((dv, dout), lambda i: (0, 0)),
            pl.BlockSpec((de, dout), lambda i: (0, 0)),
            pl.BlockSpec((1, dout), lambda i: (0, 0)),
            pl.BlockSpec((dout, dout), lambda i: (0, 0)),
            pl.BlockSpec((1, dout), lambda i: (0, 0)),
            pl.BlockSpec((dout, dout), lambda i: (0, 0)),
            pl.BlockSpec((1, dout), lambda i: (0, 0)),
        ],
        out_specs=pl.BlockSpec((blk, dout), lambda i: (i, 0)),
        out_shape=jax.ShapeDtypeStruct((total, dout), jnp.float32),
    )(v2, a2, w1v, w1a, b1[None, :], W2, b2[None, :], W3, b3[None, :])

    return h.reshape(b_dim, n_nodes, dout)


# speedup vs baseline: 7.8823x; 1.0665x over previous
import functools

import jax
import jax.numpy as jnp
from jax import lax
from jax.experimental import pallas as pl
from jax.experimental.pallas import tpu as pltpu
from jax.experimental.pallas import tpu_sc as plsc

_NUM_TILES = 16
_CHUNK = 80


def _scatter_add_sc(E_n, idx_main, idx_tail, n_nodes):
    b_dim, _, blocks_per_tile, cpb, chunk = idx_main.shape
    chunks_per_tile = blocks_per_tile * cpb + 1
    d = E_n.shape[-1]
    node_stride = (n_nodes // _NUM_TILES) // 8 * 8
    node_size = n_nodes - (_NUM_TILES - 1) * node_stride

    mesh = plsc.VectorSubcoreMesh(core_axis_name="c", subcore_axis_name="s")

    edges_per_tile = chunks_per_tile * chunk
    block = cpb * chunk

    @functools.partial(
        pl.kernel,
        mesh=mesh,
        out_type=jax.ShapeDtypeStruct((b_dim, n_nodes, d), jnp.float32),
        scratch_types=[
            pltpu.VMEM((block, d), jnp.float32),
            pltpu.VMEM((block, d), jnp.float32),
            pltpu.VMEM((cpb, chunk), jnp.int32),
            pltpu.VMEM((cpb, chunk), jnp.int32),
            pltpu.VMEM_SHARED((n_nodes, d), jnp.float32),
            pltpu.SemaphoreType.DMA,
            pltpu.SemaphoreType.DMA,
        ],
    )
    def scat(en_hbm, idxm_hbm, idxt_hbm, out_hbm, eb_a, eb_b, ib_a, ib_b,
             agg_sh, gsem_a, gsem_b):
        b = lax.axis_index("c")
        s = lax.axis_index("s")
        base_edge = s * edges_per_tile
        base_node = s * node_stride

        def gather(j, eb, ib, sem):
            e_cp = pltpu.make_async_copy(
                en_hbm.at[b, pl.ds(base_edge + j * block, block)], eb, sem)
            i_cp = pltpu.make_async_copy(idxm_hbm.at[b, s, j], ib, sem)
            return e_cp, i_cp

        def start(cps):
            cps[0].start()
            cps[1].start()

        def wait(cps):
            cps[0].wait()
            cps[1].wait()

        def scatter(eb, ib):
            for k in range(cpb):
                pltpu.sync_copy(eb.at[pl.ds(k * chunk, chunk)],
                                agg_sh.at[ib.at[k]], add=True)

        zeros16 = jnp.zeros((16,), jnp.float32)

        def zrow(r, carry):
            for k in range(d // 16):
                eb_a[r, pl.ds(k * 16, 16)] = zeros16
            return carry

        lax.fori_loop(0, block, zrow, 0)

        full = node_size // block
        for i in range(full):
            pltpu.sync_copy(eb_a, agg_sh.at[pl.ds(base_node + i * block, block)])
        rem = node_size - full * block
        if rem:
            pltpu.sync_copy(eb_a.at[pl.ds(0, rem)],
                            agg_sh.at[pl.ds(base_node + full * block, rem)])

        plsc.subcore_barrier()

        start(gather(0, eb_a, ib_a, gsem_a))

        def body(i, carry):
            g = 2 * i
            start(gather(g + 1, eb_b, ib_b, gsem_b))
            wait(gather(g, eb_a, ib_a, gsem_a))
            scatter(eb_a, ib_a)
            start(gather(g + 2, eb_a, ib_a, gsem_a))
            wait(gather(g + 1, eb_b, ib_b, gsem_b))
            scatter(eb_b, ib_b)
            return carry

        lax.fori_loop(0, (blocks_per_tile - 2) // 2, body, 0)

        g_last = blocks_per_tile - 2
        start(gather(g_last + 1, eb_b, ib_b, gsem_b))
        wait(gather(g_last, eb_a, ib_a, gsem_a))
        scatter(eb_a, ib_a)
        wait(gather(g_last + 1, eb_b, ib_b, gsem_b))
        scatter(eb_b, ib_b)

        tail_e = base_edge + blocks_per_tile * block
        pltpu.sync_copy(en_hbm.at[b, pl.ds(tail_e, chunk)],
                        eb_a.at[pl.ds(0, chunk)])
        pltpu.sync_copy(idxt_hbm.at[b, s], ib_a.at[0])
        pltpu.sync_copy(eb_a.at[pl.ds(0, chunk)],
                        agg_sh.at[ib_a.at[0]], add=True)

        plsc.subcore_barrier()

        pltpu.sync_copy(agg_sh.at[pl.ds(base_node, node_size)],
                        out_hbm.at[b, pl.ds(base_node, node_size)])

    return scat


def _mlp_body(v_ref, a_ref, w1v_ref, w1a_ref, b1_ref, w2_ref, b2_ref,
              w3_ref, b3_ref, o_ref):
    h = jnp.dot(v_ref[...], w1v_ref[...], preferred_element_type=jnp.float32)
    h = h + jnp.dot(a_ref[...], w1a_ref[...], preferred_element_type=jnp.float32)
    h = jnp.maximum(h + b1_ref[...], 0.0)
    h = jnp.maximum(
        jnp.dot(h, w2_ref[...], preferred_element_type=jnp.float32) + b2_ref[...], 0.0)
    o_ref[...] = jnp.maximum(
        jnp.dot(h, w3_ref[...], preferred_element_type=jnp.float32) + b3_ref[...], 0.0)


def kernel(V, E_n, R_r, W1, b1, W2, b2, W3, b3):
    b_dim, n_nodes, dv = V.shape
    e_edges, de = E_n.shape[1], E_n.shape[2]
    dout = W1.shape[1]

    idx = R_r[..., 0].astype(jnp.int32)
    cpt = e_edges // (_NUM_TILES * _CHUNK)
    idx4 = idx.reshape(b_dim, _NUM_TILES, cpt, _CHUNK)
    idx_main = idx4[:, :, :cpt - 1].reshape(
        b_dim, _NUM_TILES, (cpt - 1) // 2, 2, _CHUNK)
    idx_tail = idx4[:, :, cpt - 1]

    agg = _scatter_add_sc(E_n, idx_main, idx_tail, n_nodes)(
        E_n, idx_main, idx_tail)

    total = b_dim * n_nodes
    blk = 2000
    v2 = V.reshape(total, dv)
    a2 = agg.reshape(total, de)
    w1v, w1a = W1[:dv], W1[dv:]

    h = pl.pallas_call(
        _mlp_body,
        grid=(total // blk,),
        in_specs=[
            pl.BlockSpec((blk, dv), lambda i: (i, 0)),
            pl.BlockSpec((blk, de), lambda i: (i, 0)),
            pl.BlockSpec((dv, dout), lambda i: (0, 0)),
            pl.BlockSpec((de, dout), lambda i: (0, 0)),
            pl.BlockSpec((1, dout), lambda i: (0, 0)),
            pl.BlockSpec((dout, dout), lambda i: (0, 0)),
            pl.BlockSpec((1, dout), lambda i: (0, 0)),
            pl.BlockSpec((dout, dout), lambda i: (0, 0)),
            pl.BlockSpec((1, dout), lambda i: (0, 0)),
        ],
        out_specs=pl.BlockSpec((blk, dout), lambda i: (i, 0)),
        out_shape=jax.ShapeDtypeStruct((total, dout), jnp.float32),
    )(v2, a2, w1v, w1a, b1[None, :], W2, b2[None, :], W3, b3[None, :])

    return h.reshape(b_dim, n_nodes, dout)


# speedup vs baseline: 7.9274x; 1.0057x over previous
import functools

import jax
import jax.numpy as jnp
from jax import lax
from jax.experimental import pallas as pl
from jax.experimental.pallas import tpu as pltpu
from jax.experimental.pallas import tpu_sc as plsc

_NUM_TILES = 16
_CHUNK = 80


def _scatter_add_sc(E_n, idx5, n_nodes):
    b_dim, n_blocks, cpb, chunk = idx5.shape
    d = E_n.shape[-1]
    blocks_min, xtra = divmod(n_blocks, _NUM_TILES)
    node_stride = (n_nodes // _NUM_TILES) // 8 * 8
    node_size = n_nodes - (_NUM_TILES - 1) * node_stride

    mesh = plsc.VectorSubcoreMesh(core_axis_name="c", subcore_axis_name="s")

    block = cpb * chunk

    @functools.partial(
        pl.kernel,
        mesh=mesh,
        out_type=jax.ShapeDtypeStruct((b_dim, n_nodes, d), jnp.float32),
        scratch_types=[
            pltpu.VMEM((block, d), jnp.float32),
            pltpu.VMEM((block, d), jnp.float32),
            pltpu.VMEM((cpb, chunk), jnp.int32),
            pltpu.VMEM((cpb, chunk), jnp.int32),
            pltpu.VMEM_SHARED((n_nodes, d), jnp.float32),
            pltpu.SemaphoreType.DMA,
            pltpu.SemaphoreType.DMA,
        ],
    )
    def scat(en_hbm, idx_hbm, out_hbm, eb_a, eb_b, ib_a, ib_b,
             agg_sh, gsem_a, gsem_b):
        b = lax.axis_index("c")
        s = lax.axis_index("s")
        base_node = s * node_stride
        base_blk = s * blocks_min + jnp.minimum(s, xtra)
        cnt = blocks_min + jnp.where(s < xtra, 1, 0)

        def gather(g, eb, ib, sem):
            e_cp = pltpu.make_async_copy(
                en_hbm.at[b, pl.ds(g * block, block)], eb, sem)
            i_cp = pltpu.make_async_copy(idx_hbm.at[b, g], ib, sem)
            return e_cp, i_cp

        def start(cps):
            cps[0].start()
            cps[1].start()

        def wait(cps):
            cps[0].wait()
            cps[1].wait()

        def scatter(eb, ib):
            for k in range(cpb):
                pltpu.sync_copy(eb.at[pl.ds(k * chunk, chunk)],
                                agg_sh.at[ib.at[k]], add=True)

        zeros16 = jnp.zeros((16,), jnp.float32)

        def zrow(r, carry):
            for k in range(d // 16):
                eb_a[r, pl.ds(k * 16, 16)] = zeros16
            return carry

        lax.fori_loop(0, block, zrow, 0)

        full = node_size // block
        for i in range(full):
            pltpu.sync_copy(eb_a, agg_sh.at[pl.ds(base_node + i * block, block)])
        rem = node_size - full * block
        if rem:
            pltpu.sync_copy(eb_a.at[pl.ds(0, rem)],
                            agg_sh.at[pl.ds(base_node + full * block, rem)])

        plsc.subcore_barrier()

        start(gather(base_blk, eb_a, ib_a, gsem_a))

        def body(t, carry):
            @pl.when(t % 2 == 0)
            def _():
                start(gather(base_blk + t + 1, eb_b, ib_b, gsem_b))
                wait(gather(base_blk + t, eb_a, ib_a, gsem_a))
                scatter(eb_a, ib_a)

            @pl.when(t % 2 == 1)
            def _():
                start(gather(base_blk + t + 1, eb_a, ib_a, gsem_a))
                wait(gather(base_blk + t, eb_b, ib_b, gsem_b))
                scatter(eb_b, ib_b)

            return carry

        lax.fori_loop(0, cnt - 1, body, 0)
        last = cnt - 1

        @pl.when(last % 2 == 0)
        def _():
            wait(gather(base_blk + last, eb_a, ib_a, gsem_a))
            scatter(eb_a, ib_a)

        @pl.when(last % 2 == 1)
        def _():
            wait(gather(base_blk + last, eb_b, ib_b, gsem_b))
            scatter(eb_b, ib_b)

        plsc.subcore_barrier()

        pltpu.sync_copy(agg_sh.at[pl.ds(base_node, node_size)],
                        out_hbm.at[b, pl.ds(base_node, node_size)])

    return scat


def _mlp_body(v_ref, a_ref, w1v_ref, w1a_ref, b1_ref, w2_ref, b2_ref,
              w3_ref, b3_ref, o_ref):
    h = jnp.dot(v_ref[...], w1v_ref[...], preferred_element_type=jnp.float32)
    h = h + jnp.dot(a_ref[...], w1a_ref[...], preferred_element_type=jnp.float32)
    h = jnp.maximum(h + b1_ref[...], 0.0)
    h = jnp.maximum(
        jnp.dot(h, w2_ref[...], preferred_element_type=jnp.float32) + b2_ref[...], 0.0)
    o_ref[...] = jnp.maximum(
        jnp.dot(h, w3_ref[...], preferred_element_type=jnp.float32) + b3_ref[...], 0.0)


def kernel(V, E_n, R_r, W1, b1, W2, b2, W3, b3):
    b_dim, n_nodes, dv = V.shape
    e_edges, de = E_n.shape[1], E_n.shape[2]
    dout = W1.shape[1]

    idx5 = R_r.reshape(b_dim, e_edges // (2 * _CHUNK), 2, _CHUNK)

    agg = _scatter_add_sc(E_n, idx5, n_nodes)(E_n, idx5)

    total = b_dim * n_nodes
    blk = 2000
    v2 = V.reshape(total, dv)
    a2 = agg.reshape(total, de)
    w1v, w1a = W1[:dv], W1[dv:]

    h = pl.pallas_call(
        _mlp_body,
        grid=(total // blk,),
        in_specs=[
            pl.BlockSpec((blk, dv), lambda i: (i, 0)),
            pl.BlockSpec((blk, de), lambda i: (i, 0)),
            pl.BlockSpec((dv, dout), lambda i: (0, 0)),
            pl.BlockSpec((de, dout), lambda i: (0, 0)),
            pl.BlockSpec((1, dout), lambda i: (0, 0)),
            pl.BlockSpec((dout, dout), lambda i: (0, 0)),
            pl.BlockSpec((1, dout), lambda i: (0, 0)),
            pl.BlockSpec((dout, dout), lambda i: (0, 0)),
            pl.BlockSpec((1, dout), lambda i: (0, 0)),
        ],
        out_specs=pl.BlockSpec((blk, dout), lambda i: (i, 0)),
        out_shape=jax.ShapeDtypeStruct((total, dout), jnp.float32),
    )(v2, a2, w1v, w1a, b1[None, :], W2, b2[None, :], W3, b3[None, :])

    return h.reshape(b_dim, n_nodes, dout)


# speedup vs baseline: 8.3137x; 1.0487x over previous
import functools

import jax
import jax.numpy as jnp
from jax import lax
from jax.experimental import pallas as pl
from jax.experimental.pallas import tpu as pltpu
from jax.experimental.pallas import tpu_sc as plsc

_NUM_TILES = 16
_CHUNK = 80


def _scatter_add_sc(E_n, idx4, n_nodes):
    b_dim, n_chunks, _, chunk = idx4.shape
    d = E_n.shape[-1]
    chunks_per_tile = n_chunks // _NUM_TILES
    node_stride = (n_nodes // _NUM_TILES) // 8 * 8
    node_size = n_nodes - (_NUM_TILES - 1) * node_stride

    mesh = plsc.VectorSubcoreMesh(core_axis_name="c", subcore_axis_name="s")

    nbuf = 4

    @functools.partial(
        pl.kernel,
        mesh=mesh,
        out_type=jax.ShapeDtypeStruct((b_dim, n_nodes, d), jnp.float32),
        scratch_types=[
            pltpu.VMEM((nbuf, chunk, d), jnp.float32),
            pltpu.VMEM((nbuf, 1, chunk), jnp.int32),
            pltpu.VMEM_SHARED((n_nodes, d), jnp.float32),
            pltpu.SemaphoreType.DMA,
            pltpu.SemaphoreType.DMA,
        ],
    )
    def scat(en_hbm, idx_hbm, out_hbm, eb, ib, agg_sh, gsem, ssem):
        b = lax.axis_index("c")
        s = lax.axis_index("s")
        base_node = s * node_stride
        base_chunk = s * chunks_per_tile
        base_edge = s * chunks_per_tile * chunk

        def gather(j, p):
            e_cp = pltpu.make_async_copy(
                en_hbm.at[b, pl.ds(base_edge + j * chunk, chunk)],
                eb.at[p], gsem)
            i_cp = pltpu.make_async_copy(
                idx_hbm.at[b, base_chunk + j], ib.at[p], gsem)
            return e_cp, i_cp

        def g_start(cps):
            cps[0].start()
            cps[1].start()

        def g_wait(cps):
            cps[0].wait()
            cps[1].wait()

        def scatter(p):
            return pltpu.make_async_copy(
                eb.at[p], agg_sh.at[ib.at[p, 0]], ssem)

        zeros16 = jnp.zeros((16,), jnp.float32)

        def zrow(r, carry):
            for k in range(d // 16):
                eb[0, r, pl.ds(k * 16, 16)] = zeros16
            return carry

        lax.fori_loop(0, chunk, zrow, 0)

        for i in range(node_size // chunk):
            pltpu.sync_copy(eb.at[0],
                            agg_sh.at[pl.ds(base_node + i * chunk, chunk)])

        plsc.subcore_barrier()

        def step(j, with_wait, with_start):
            p = j % nbuf
            g_wait(gather(j, p))
            scatter(p).start(add=True)
            if with_wait:
                pw = (j - 2) % nbuf
                scatter(pw).wait()
            if with_start:
                g_start(gather(j + 2, (j + 2) % nbuf))

        g_start(gather(0, 0))
        g_start(gather(1, 1))
        step(0, False, True)
        step(1, False, True)

        def body(i, carry):
            j = nbuf * i + 2
            for k in range(nbuf):
                p = (2 + k) % nbuf
                jj = j + k
                g_wait(gather(jj, p))
                scatter(p).start(add=True)
                scatter((p - 2) % nbuf).wait()
                g_start(gather(jj + 2, (p + 2) % nbuf))
            return carry

        main_iters = (chunks_per_tile - 2 - 5) // nbuf
        lax.fori_loop(0, main_iters, body, 0)
        for j in range(2 + nbuf * main_iters, chunks_per_tile):
            step(j, True, j + 2 < chunks_per_tile)
        scatter((chunks_per_tile - 2) % nbuf).wait()
        scatter((chunks_per_tile - 1) % nbuf).wait()

        plsc.subcore_barrier()

        pltpu.sync_copy(agg_sh.at[pl.ds(base_node, node_size)],
                        out_hbm.at[b, pl.ds(base_node, node_size)])

    return scat


def _mlp_body(v_ref, a_ref, w1v_ref, w1a_ref, b1_ref, w2_ref, b2_ref,
              w3_ref, b3_ref, o_ref):
    h = jnp.dot(v_ref[...], w1v_ref[...], preferred_element_type=jnp.float32)
    h = h + jnp.dot(a_ref[...], w1a_ref[...], preferred_element_type=jnp.float32)
    h = jnp.maximum(h + b1_ref[...], 0.0)
    h = jnp.maximum(
        jnp.dot(h, w2_ref[...], preferred_element_type=jnp.float32) + b2_ref[...], 0.0)
    o_ref[...] = jnp.maximum(
        jnp.dot(h, w3_ref[...], preferred_element_type=jnp.float32) + b3_ref[...], 0.0)


def kernel(V, E_n, R_r, W1, b1, W2, b2, W3, b3):
    b_dim, n_nodes, dv = V.shape
    e_edges, de = E_n.shape[1], E_n.shape[2]
    dout = W1.shape[1]

    idx4 = R_r.reshape(b_dim, e_edges // _CHUNK, 1, _CHUNK)

    agg = _scatter_add_sc(E_n, idx4, n_nodes)(E_n, idx4)

    total = b_dim * n_nodes
    blk = 2000
    v2 = V.reshape(total, dv)
    a2 = agg.reshape(total, de)
    w1v, w1a = W1[:dv], W1[dv:]

    h = pl.pallas_call(
        _mlp_body,
        grid=(total // blk,),
        in_specs=[
            pl.BlockSpec((blk, dv), lambda i: (i, 0)),
            pl.BlockSpec((blk, de), lambda i: (i, 0)),
            pl.BlockSpec((dv, dout), lambda i: (0, 0)),
            pl.BlockSpec((de, dout), lambda i: (0, 0)),
            pl.BlockSpec((1, dout), lambda i: (0, 0)),
            pl.BlockSpec((dout, dout), lambda i: (0, 0)),
            pl.BlockSpec((1, dout), lambda i: (0, 0)),
            pl.BlockSpec((dout, dout), lambda i: (0, 0)),
            pl.BlockSpec((1, dout), lambda i: (0, 0)),
        ],
        out_specs=pl.BlockSpec((blk, dout), lambda i: (i, 0)),
        out_shape=jax.ShapeDtypeStruct((total, dout), jnp.float32),
    )(v2, a2, w1v, w1a, b1[None, :], W2, b2[None, :], W3, b3[None, :])

    return h.reshape(b_dim, n_nodes, dout)


# speedup vs baseline: 8.4895x; 1.0211x over previous
import functools

import jax
import jax.numpy as jnp
from jax import lax
from jax.experimental import pallas as pl
from jax.experimental.pallas import tpu as pltpu
from jax.experimental.pallas import tpu_sc as plsc

_NUM_TILES = 16
_CHUNK = 80


def _scatter_add_sc(E_n, idx4, n_nodes):
    b_dim, n_chunks, _, chunk = idx4.shape
    d = E_n.shape[-1]
    chunks_per_tile = n_chunks // _NUM_TILES
    node_stride = (n_nodes // _NUM_TILES) // 8 * 8
    node_size = n_nodes - (_NUM_TILES - 1) * node_stride

    mesh = plsc.VectorSubcoreMesh(core_axis_name="c", subcore_axis_name="s")

    nbuf = 4

    @functools.partial(
        pl.kernel,
        mesh=mesh,
        out_type=jax.ShapeDtypeStruct((b_dim, n_nodes, d), jnp.float32),
        scratch_types=[
            pltpu.VMEM((nbuf, chunk, d), jnp.float32),
            pltpu.VMEM((nbuf, 1, chunk), jnp.int32),
            pltpu.VMEM_SHARED((n_nodes, d), jnp.float32),
            pltpu.SemaphoreType.DMA,
            pltpu.SemaphoreType.DMA,
        ],
    )
    def scat(en_hbm, idx_hbm, out_hbm, eb, ib, agg_sh, gsem, ssem):
        b = lax.axis_index("c")
        s = lax.axis_index("s")
        base_node = s * node_stride
        base_chunk = s * chunks_per_tile
        base_edge = s * chunks_per_tile * chunk

        def gather(j, p):
            e_cp = pltpu.make_async_copy(
                en_hbm.at[b, pl.ds(base_edge + j * chunk, chunk)],
                eb.at[p], gsem)
            i_cp = pltpu.make_async_copy(
                idx_hbm.at[b, base_chunk + j], ib.at[p], gsem)
            return e_cp, i_cp

        def g_start(cps):
            cps[0].start()
            cps[1].start()

        def g_wait(cps):
            cps[0].wait()
            cps[1].wait()

        def scatter(p):
            return pltpu.make_async_copy(
                eb.at[p], agg_sh.at[ib.at[p, 0]], ssem)

        zeros16 = jnp.zeros((16,), jnp.float32)

        def zrow(r, carry):
            for k in range(d // 16):
                eb[0, r, pl.ds(k * 16, 16)] = zeros16
            return carry

        lax.fori_loop(0, chunk, zrow, 0)

        for i in range(node_size // chunk):
            pltpu.sync_copy(eb.at[0],
                            agg_sh.at[pl.ds(base_node + i * chunk, chunk)])

        plsc.subcore_barrier()

        def step(j, with_wait, with_start):
            p = j % nbuf
            g_wait(gather(j, p))
            scatter(p).start(add=True)
            if with_wait:
                pw = (j - 2) % nbuf
                scatter(pw).wait()
            if with_start:
                g_start(gather(j + 2, (j + 2) % nbuf))

        g_start(gather(0, 0))
        g_start(gather(1, 1))
        step(0, False, True)
        step(1, False, True)

        def body(i, carry):
            j = nbuf * i + 2
            for k in range(nbuf):
                p = (2 + k) % nbuf
                jj = j + k
                g_wait(gather(jj, p))
                scatter(p).start(add=True)
                scatter((p - 2) % nbuf).wait()
                g_start(gather(jj + 2, (p + 2) % nbuf))
            return carry

        main_iters = (chunks_per_tile - 2 - 5) // nbuf
        lax.fori_loop(0, main_iters, body, 0)
        for j in range(2 + nbuf * main_iters, chunks_per_tile):
            step(j, True, j + 2 < chunks_per_tile)
        scatter((chunks_per_tile - 2) % nbuf).wait()
        scatter((chunks_per_tile - 1) % nbuf).wait()

        plsc.subcore_barrier()

        pltpu.sync_copy(agg_sh.at[pl.ds(base_node, node_size)],
                        out_hbm.at[b, pl.ds(base_node, node_size)])

    return scat


def _mlp_body(v_ref, a_ref, w1v_ref, w1a_ref, b1_ref, w2_ref, b2_ref,
              w3_ref, b3_ref, o_ref):
    h = jnp.dot(v_ref[...].astype(jnp.float32), w1v_ref[...],
                preferred_element_type=jnp.float32)
    h = h + jnp.dot(a_ref[...], w1a_ref[...], preferred_element_type=jnp.float32)
    h = jnp.maximum(h + b1_ref[...], 0.0)
    h = jnp.maximum(
        jnp.dot(h, w2_ref[...], preferred_element_type=jnp.float32) + b2_ref[...], 0.0)
    o_ref[...] = jnp.maximum(
        jnp.dot(h, w3_ref[...], preferred_element_type=jnp.float32) + b3_ref[...], 0.0)


def kernel(V, E_n, R_r, W1, b1, W2, b2, W3, b3):
    b_dim, n_nodes, dv = V.shape
    e_edges, de = E_n.shape[1], E_n.shape[2]
    dout = W1.shape[1]

    idx4 = R_r.reshape(b_dim, e_edges // _CHUNK, 1, _CHUNK)

    agg = _scatter_add_sc(E_n, idx4, n_nodes)(E_n, idx4)

    total = b_dim * n_nodes
    blk = 4000
    v2 = V.reshape(total, dv).astype(jnp.bfloat16)
    a2 = agg.reshape(total, de)
    w1v, w1a = W1[:dv], W1[dv:]

    h = pl.pallas_call(
        _mlp_body,
        grid=(total // blk,),
        in_specs=[
            pl.BlockSpec((blk, dv), lambda i: (i, 0)),
            pl.BlockSpec((blk, de), lambda i: (i, 0)),
            pl.BlockSpec((dv, dout), lambda i: (0, 0)),
            pl.BlockSpec((de, dout), lambda i: (0, 0)),
            pl.BlockSpec((1, dout), lambda i: (0, 0)),
            pl.BlockSpec((dout, dout), lambda i: (0, 0)),
            pl.BlockSpec((1, dout), lambda i: (0, 0)),
            pl.BlockSpec((dout, dout), lambda i: (0, 0)),
            pl.BlockSpec((1, dout), lambda i: (0, 0)),
        ],
        out_specs=pl.BlockSpec((blk, dout), lambda i: (i, 0)),
        out_shape=jax.ShapeDtypeStruct((total, dout), jnp.float32),
    )(v2, a2, w1v, w1a, b1[None, :], W2, b2[None, :], W3, b3[None, :])

    return h.reshape(b_dim, n_nodes, dout)


# speedup vs baseline: 8.9297x; 1.0519x over previous
import functools

import jax
import jax.numpy as jnp
from jax import lax
from jax.experimental import pallas as pl
from jax.experimental.pallas import tpu as pltpu
from jax.experimental.pallas import tpu_sc as plsc

_NUM_TILES = 16
_CHUNK = 80


def _scatter_add_sc(E_n, idx4, n_nodes):
    b_dim, n_chunks, _, chunk = idx4.shape
    d = E_n.shape[-1]
    chunks_per_tile = n_chunks // _NUM_TILES
    node_stride = (n_nodes // _NUM_TILES) // 8 * 8
    node_size = n_nodes - (_NUM_TILES - 1) * node_stride

    mesh = plsc.VectorSubcoreMesh(core_axis_name="c", subcore_axis_name="s")

    nbuf = 4

    @functools.partial(
        pl.kernel,
        mesh=mesh,
        out_type=jax.ShapeDtypeStruct((b_dim, n_nodes, d), jnp.float32),
        scratch_types=[
            pltpu.VMEM((nbuf, chunk, d), jnp.float32),
            pltpu.VMEM((nbuf, 1, chunk), jnp.int32),
            pltpu.VMEM_SHARED((n_nodes, d), jnp.float32),
            pltpu.SemaphoreType.DMA,
            pltpu.SemaphoreType.DMA,
        ],
    )
    def scat(en_hbm, idx_hbm, out_hbm, eb, ib, agg_sh, gsem, ssem):
        b = lax.axis_index("c")
        s = lax.axis_index("s")
        base_node = s * node_stride
        base_chunk = s * chunks_per_tile
        base_edge = s * chunks_per_tile * chunk

        def gather(j, p):
            e_cp = pltpu.make_async_copy(
                en_hbm.at[b, pl.ds(base_edge + j * chunk, chunk)],
                eb.at[p], gsem)
            i_cp = pltpu.make_async_copy(
                idx_hbm.at[b, base_chunk + j], ib.at[p], gsem)
            return e_cp, i_cp

        def g_start(cps):
            cps[0].start()
            cps[1].start()

        def g_wait(cps):
            cps[0].wait()
            cps[1].wait()

        def scatter(p):
            return pltpu.make_async_copy(
                eb.at[p], agg_sh.at[ib.at[p, 0]], ssem)

        zeros16 = jnp.zeros((16,), jnp.float32)

        def zrow(r, carry):
            for k in range(d // 16):
                eb[0, r, pl.ds(k * 16, 16)] = zeros16
            return carry

        lax.fori_loop(0, chunk, zrow, 0)

        for i in range(node_size // chunk):
            pltpu.sync_copy(eb.at[0],
                            agg_sh.at[pl.ds(base_node + i * chunk, chunk)])

        plsc.subcore_barrier()

        def step(j, with_wait, with_start):
            p = j % nbuf
            g_wait(gather(j, p))
            scatter(p).start(add=True)
            if with_wait:
                pw = (j - 1) % nbuf
                scatter(pw).wait()
            if with_start:
                g_start(gather(j + 3, (j + 3) % nbuf))

        g_start(gather(0, 0))
        g_start(gather(1, 1))
        g_start(gather(2, 2))
        step(0, False, True)
        for j in (1, 2, 3):
            step(j, True, True)

        def body(i, carry):
            j = nbuf * i + 4
            for k in range(nbuf):
                p = (4 + k) % nbuf
                jj = j + k
                g_wait(gather(jj, p))
                scatter(p).start(add=True)
                scatter((p - 1) % nbuf).wait()
                g_start(gather(jj + 3, (p + 3) % nbuf))
            return carry

        main_iters = (chunks_per_tile - 4 - 5) // nbuf
        lax.fori_loop(0, main_iters, body, 0)
        for j in range(4 + nbuf * main_iters, chunks_per_tile):
            step(j, True, j + 3 < chunks_per_tile)
        scatter((chunks_per_tile - 1) % nbuf).wait()

        plsc.subcore_barrier()

        pltpu.sync_copy(agg_sh.at[pl.ds(base_node, node_size)],
                        out_hbm.at[b, pl.ds(base_node, node_size)])

    return scat


def _mlp_body(v_ref, a_ref, w1v_ref, w1a_ref, b1_ref, w2_ref, b2_ref,
              w3_ref, b3_ref, o_ref):
    h = jnp.dot(v_ref[...].astype(jnp.float32), w1v_ref[...],
                preferred_element_type=jnp.float32)
    h = h + jnp.dot(a_ref[...], w1a_ref[...], preferred_element_type=jnp.float32)
    h = jnp.maximum(h + b1_ref[...], 0.0)
    h = jnp.maximum(
        jnp.dot(h, w2_ref[...], preferred_element_type=jnp.float32) + b2_ref[...], 0.0)
    o_ref[...] = jnp.maximum(
        jnp.dot(h, w3_ref[...], preferred_element_type=jnp.float32) + b3_ref[...], 0.0)


def kernel(V, E_n, R_r, W1, b1, W2, b2, W3, b3):
    b_dim, n_nodes, dv = V.shape
    e_edges, de = E_n.shape[1], E_n.shape[2]
    dout = W1.shape[1]

    idx4 = R_r.reshape(b_dim, e_edges // _CHUNK, 1, _CHUNK)

    agg = _scatter_add_sc(E_n, idx4, n_nodes)(E_n, idx4)

    total = b_dim * n_nodes
    blk = 4000
    v2 = V.reshape(total, dv).astype(jnp.bfloat16)
    a2 = agg.reshape(total, de)
    w1v, w1a = W1[:dv], W1[dv:]

    h = pl.pallas_call(
        _mlp_body,
        grid=(total // blk,),
        in_specs=[
            pl.BlockSpec((blk, dv), lambda i: (i, 0)),
            pl.BlockSpec((blk, de), lambda i: (i, 0)),
            pl.BlockSpec((dv, dout), lambda i: (0, 0)),
            pl.BlockSpec((de, dout), lambda i: (0, 0)),
            pl.BlockSpec((1, dout), lambda i: (0, 0)),
            pl.BlockSpec((dout, dout), lambda i: (0, 0)),
            pl.BlockSpec((1, dout), lambda i: (0, 0)),
            pl.BlockSpec((dout, dout), lambda i: (0, 0)),
            pl.BlockSpec((1, dout), lambda i: (0, 0)),
        ],
        out_specs=pl.BlockSpec((blk, dout), lambda i: (i, 0)),
        out_shape=jax.ShapeDtypeStruct((total, dout), jnp.float32),
    )(v2, a2, w1v, w1a, b1[None, :], W2, b2[None, :], W3, b3[None, :])

    return h.reshape(b_dim, n_nodes, dout)
